# Initial kernel scaffold; baseline (speedup 1.0000x reference)
import jax
import jax.numpy as jnp
from jax.experimental import pallas as pl


def kernel(x, adj0, adj1, adj2, W, a, Ws, bs, q, Wo, bo):
    raise NotImplementedError("write your pallas kernel here")


# speedup vs baseline: 1.8879x; 1.8879x over previous
import jax
import jax.numpy as jnp
from jax.experimental import pallas as pl

N = 2048
FEAT = 128
HID = 32
HEADS = 8
CLASSES = 16
QV = 128
ALPHA = 0.2
NEG = -9e15

BR = 256
NB = N // BR


def _prep_kernel(x_ref, W_ref, a_ref, h_ref, f1_ref, f2_ref):
    h = jnp.dot(x_ref[...], W_ref[0], preferred_element_type=jnp.float32)
    a1 = a_ref[0, :HID, :]
    a2 = a_ref[0, HID:, :]
    h_ref[0] = h
    f1_ref[0] = jnp.dot(h, a1, preferred_element_type=jnp.float32)
    f2_ref[0] = jax.lax.dot_general(
        a2, h, (((0,), (1,)), ((), ())), preferred_element_type=jnp.float32)


def _attn_kernel(adj0_ref, adj1_ref, adj2_ref, hall_ref, f1_ref, f2_ref,
                 z0_ref, z1_ref, z2_ref):
    i = pl.program_id(0)
    hd = pl.program_id(1)
    hh = hall_ref[hd]
    f1b = f1_ref[hd, pl.ds(i * BR, BR), :]
    f2r = f2_ref[hd]
    s = f1b + f2r
    e_raw = jnp.where(s >= 0, s, ALPHA * s)
    for adj_ref, z_ref in ((adj0_ref, z0_ref), (adj1_ref, z1_ref),
                           (adj2_ref, z2_ref)):
        e = jnp.where(adj_ref[...] > 0, e_raw, NEG)
        m = jnp.max(e, axis=1, keepdims=True)
        p = jnp.exp(e - m)
        denom = jnp.sum(p, axis=1, keepdims=True)
        num = jnp.dot(p, hh, preferred_element_type=jnp.float32)
        att = num / denom
        z_ref[0] = jnp.where(att > 0, att, jnp.exp(att) - 1.0)


def _sem_kernel(z0_ref, z1_ref, z2_ref, Ws_ref, bs_ref, q_ref, Wo_ref,
                bo_ref, out_ref):
    wbars = []
    ys = []
    for z_ref in (z0_ref, z1_ref, z2_ref):
        t = jnp.zeros((N, QV), dtype=jnp.float32)
        y = jnp.zeros((N, CLASSES), dtype=jnp.float32)
        for hd in range(HEADS):
            zh = z_ref[hd]
            t = t + jnp.dot(zh, Ws_ref[hd], preferred_element_type=jnp.float32)
            y = y + jnp.dot(zh, Wo_ref[hd], preferred_element_type=jnp.float32)
        w = jnp.dot(jnp.tanh(t + bs_ref[...]), q_ref[...],
                    preferred_element_type=jnp.float32)
        wbars.append(jnp.sum(w) / N)
        ys.append(y)
    m = jnp.maximum(jnp.maximum(wbars[0], wbars[1]), wbars[2])
    e0 = jnp.exp(wbars[0] - m)
    e1 = jnp.exp(wbars[1] - m)
    e2 = jnp.exp(wbars[2] - m)
    denom = e0 + e1 + e2
    out_ref[...] = (e0 * ys[0] + e1 * ys[1] + e2 * ys[2]) / denom + bo_ref[...]


def kernel(x, adj0, adj1, adj2, W, a, Ws, bs, q, Wo, bo):
    a3 = a.reshape(HEADS, 2 * HID, 1)
    hall, f1, f2 = pl.pallas_call(
        _prep_kernel,
        grid=(HEADS,),
        in_specs=[
            pl.BlockSpec((N, FEAT), lambda h: (0, 0)),
            pl.BlockSpec((1, FEAT, HID), lambda h: (h, 0, 0)),
            pl.BlockSpec((1, 2 * HID, 1), lambda h: (h, 0, 0)),
        ],
        out_specs=[
            pl.BlockSpec((1, N, HID), lambda h: (h, 0, 0)),
            pl.BlockSpec((1, N, 1), lambda h: (h, 0, 0)),
            pl.BlockSpec((1, 1, N), lambda h: (h, 0, 0)),
        ],
        out_shape=[
            jax.ShapeDtypeStruct((HEADS, N, HID), jnp.float32),
            jax.ShapeDtypeStruct((HEADS, N, 1), jnp.float32),
            jax.ShapeDtypeStruct((HEADS, 1, N), jnp.float32),
        ],
    )(x, W, a3)

    adj_spec = pl.BlockSpec((BR, N), lambda i, h: (i, 0))
    z_spec = pl.BlockSpec((1, BR, HID), lambda i, h: (h, i, 0))
    z_shape = jax.ShapeDtypeStruct((HEADS, N, HID), jnp.float32)
    z0, z1, z2 = pl.pallas_call(
        _attn_kernel,
        grid=(NB, HEADS),
        in_specs=[
            adj_spec, adj_spec, adj_spec,
            pl.BlockSpec((HEADS, N, HID), lambda i, h: (0, 0, 0)),
            pl.BlockSpec((HEADS, N, 1), lambda i, h: (0, 0, 0)),
            pl.BlockSpec((HEADS, 1, N), lambda i, h: (0, 0, 0)),
        ],
        out_specs=[z_spec, z_spec, z_spec],
        out_shape=[z_shape, z_shape, z_shape],
    )(adj0, adj1, adj2, hall, f1, f2)

    out = pl.pallas_call(
        _sem_kernel,
        out_shape=jax.ShapeDtypeStruct((N, CLASSES), jnp.float32),
    )(z0, z1, z2,
      Ws.reshape(HEADS, HID, QV), bs.reshape(1, QV), q.reshape(QV, 1),
      Wo.reshape(HEADS, HID, CLASSES), bo.reshape(1, CLASSES))
    return out


# speedup vs baseline: 2.1484x; 1.1380x over previous
import jax
import jax.numpy as jnp
from jax.experimental import pallas as pl

N = 2048
FEAT = 128
HID = 32
HEADS = 8
CLASSES = 16
QV = 128
ALPHA = 0.2
NEG = -9e15

BR = 256
NB = N // BR


def _prep_kernel(x_ref, W_ref, a_ref, h_ref, f1_ref, f2_ref):
    h = jnp.dot(x_ref[...], W_ref[0], preferred_element_type=jnp.float32)
    a1 = a_ref[0, :HID, :]
    a2 = a_ref[0, HID:, :]
    h_ref[0] = h
    f1_ref[0] = jnp.dot(h, a1, preferred_element_type=jnp.float32)
    f2_ref[0] = jax.lax.dot_general(
        a2, h, (((0,), (1,)), ((), ())), preferred_element_type=jnp.float32)


def _attn_kernel(adj0_ref, adj1_ref, adj2_ref, hall_ref, f1_ref, f2_ref,
                 z0_ref, z1_ref, z2_ref):
    i = pl.program_id(0)
    hd = pl.program_id(1)
    hh = hall_ref[hd]
    f1b = f1_ref[hd, pl.ds(i * BR, BR), :]
    f2r = f2_ref[hd]
    mb = f1b + jnp.max(f2r)
    mb = jnp.where(mb >= 0, mb, ALPHA * mb)
    s = f1b + f2r
    e_raw = jnp.where(s >= 0, s, ALPHA * s)
    pshared = jnp.exp(e_raw - mb)
    for adj_ref, z_ref in ((adj0_ref, z0_ref), (adj1_ref, z1_ref),
                           (adj2_ref, z2_ref)):
        p = pshared * adj_ref[...].astype(jnp.float32)
        denom = jnp.maximum(jnp.sum(p, axis=1, keepdims=True), 1e-38)
        num = jnp.dot(p, hh, preferred_element_type=jnp.float32)
        att = num / denom
        z_ref[0] = jnp.where(att > 0, att, jnp.exp(att) - 1.0)


def _sem_kernel(z0_ref, z1_ref, z2_ref, Ws_ref, bs_ref, q_ref, Wo_ref,
                bo_ref, out_ref):
    wbars = []
    ys = []
    for z_ref in (z0_ref, z1_ref, z2_ref):
        t = jnp.zeros((N, QV), dtype=jnp.float32)
        y = jnp.zeros((N, CLASSES), dtype=jnp.float32)
        for hd in range(HEADS):
            zh = z_ref[hd]
            t = t + jnp.dot(zh, Ws_ref[hd], preferred_element_type=jnp.float32)
            y = y + jnp.dot(zh, Wo_ref[hd], preferred_element_type=jnp.float32)
        w = jnp.dot(jnp.tanh(t + bs_ref[...]), q_ref[...],
                    preferred_element_type=jnp.float32)
        wbars.append(jnp.sum(w) / N)
        ys.append(y)
    m = jnp.maximum(jnp.maximum(wbars[0], wbars[1]), wbars[2])
    e0 = jnp.exp(wbars[0] - m)
    e1 = jnp.exp(wbars[1] - m)
    e2 = jnp.exp(wbars[2] - m)
    denom = e0 + e1 + e2
    out_ref[...] = (e0 * ys[0] + e1 * ys[1] + e2 * ys[2]) / denom + bo_ref[...]


def kernel(x, adj0, adj1, adj2, W, a, Ws, bs, q, Wo, bo):
    a3 = a.reshape(HEADS, 2 * HID, 1)
    hall, f1, f2 = pl.pallas_call(
        _prep_kernel,
        grid=(HEADS,),
        in_specs=[
            pl.BlockSpec((N, FEAT), lambda h: (0, 0)),
            pl.BlockSpec((1, FEAT, HID), lambda h: (h, 0, 0)),
            pl.BlockSpec((1, 2 * HID, 1), lambda h: (h, 0, 0)),
        ],
        out_specs=[
            pl.BlockSpec((1, N, HID), lambda h: (h, 0, 0)),
            pl.BlockSpec((1, N, 1), lambda h: (h, 0, 0)),
            pl.BlockSpec((1, 1, N), lambda h: (h, 0, 0)),
        ],
        out_shape=[
            jax.ShapeDtypeStruct((HEADS, N, HID), jnp.float32),
            jax.ShapeDtypeStruct((HEADS, N, 1), jnp.float32),
            jax.ShapeDtypeStruct((HEADS, 1, N), jnp.float32),
        ],
    )(x, W, a3)

    adj_spec = pl.BlockSpec((BR, N), lambda i, h: (i, 0))
    z_spec = pl.BlockSpec((1, BR, HID), lambda i, h: (h, i, 0))
    z_shape = jax.ShapeDtypeStruct((HEADS, N, HID), jnp.float32)
    z0, z1, z2 = pl.pallas_call(
        _attn_kernel,
        grid=(NB, HEADS),
        in_specs=[
            adj_spec, adj_spec, adj_spec,
            pl.BlockSpec((HEADS, N, HID), lambda i, h: (0, 0, 0)),
            pl.BlockSpec((HEADS, N, 1), lambda i, h: (0, 0, 0)),
            pl.BlockSpec((HEADS, 1, N), lambda i, h: (0, 0, 0)),
        ],
        out_specs=[z_spec, z_spec, z_spec],
        out_shape=[z_shape, z_shape, z_shape],
    )(adj0, adj1, adj2, hall, f1, f2)

    out = pl.pallas_call(
        _sem_kernel,
        out_shape=jax.ShapeDtypeStruct((N, CLASSES), jnp.float32),
    )(z0, z1, z2,
      Ws.reshape(HEADS, HID, QV), bs.reshape(1, QV), q.reshape(QV, 1),
      Wo.reshape(HEADS, HID, CLASSES), bo.reshape(1, CLASSES))
    return out


# speedup vs baseline: 2.5227x; 1.1742x over previous
import jax
import jax.numpy as jnp
from jax.experimental import pallas as pl

N = 2048
FEAT = 128
HID = 32
HEADS = 8
CLASSES = 16
QV = 128
ALPHA = 0.2
NEG = -9e15

BR = 256
NB = N // BR


def _prep_kernel(x_ref, W_ref, a_ref, hb_ref, f1_ref, f2_ref):
    h = jnp.dot(x_ref[...], W_ref[0], preferred_element_type=jnp.float32)
    a1 = a_ref[0, :HID, :]
    a2 = a_ref[0, HID:, :]
    hb_ref[0] = jnp.concatenate(
        [h, jnp.ones((N, 1), jnp.float32)], axis=1).astype(jnp.bfloat16)
    f1_ref[0] = jnp.dot(h, a1, preferred_element_type=jnp.float32)
    f2_ref[0] = jax.lax.dot_general(
        a2, h, (((0,), (1,)), ((), ())), preferred_element_type=jnp.float32)


def _attn_kernel(adj0_ref, adj1_ref, adj2_ref, hb_ref, f1_ref, f2_ref,
                 z0_ref, z1_ref, z2_ref):
    i = pl.program_id(0)
    hd = pl.program_id(1)
    hb = hb_ref[hd]
    f1b = f1_ref[hd, pl.ds(i * BR, BR), :]
    f2r = f2_ref[hd]
    mb = f1b + jnp.max(f2r)
    mb = jnp.where(mb >= 0, mb, ALPHA * mb)
    s = f1b + f2r
    e_raw = jnp.where(s >= 0, s, ALPHA * s)
    pshared = jnp.exp(e_raw - mb).astype(jnp.bfloat16)
    for adj_ref, z_ref in ((adj0_ref, z0_ref), (adj1_ref, z1_ref),
                           (adj2_ref, z2_ref)):
        p = pshared * adj_ref[...].astype(jnp.bfloat16)
        nd = jnp.dot(p, hb, preferred_element_type=jnp.float32)
        denom = jnp.maximum(nd[:, HID:], 1e-38)
        att = nd[:, :HID] / denom
        z_ref[0] = jnp.where(att > 0, att, jnp.exp(att) - 1.0)


def _sem_kernel(z0_ref, z1_ref, z2_ref, Ws_ref, bs_ref, q_ref, Wo_ref,
                bo_ref, out_ref):
    wbars = []
    ys = []
    for z_ref in (z0_ref, z1_ref, z2_ref):
        t = jnp.zeros((N, QV), dtype=jnp.float32)
        y = jnp.zeros((N, CLASSES), dtype=jnp.float32)
        for hd in range(HEADS):
            zh = z_ref[hd]
            t = t + jnp.dot(zh, Ws_ref[hd], preferred_element_type=jnp.float32)
            y = y + jnp.dot(zh, Wo_ref[hd], preferred_element_type=jnp.float32)
        w = jnp.dot(jnp.tanh(t + bs_ref[...]), q_ref[...],
                    preferred_element_type=jnp.float32)
        wbars.append(jnp.sum(w) / N)
        ys.append(y)
    m = jnp.maximum(jnp.maximum(wbars[0], wbars[1]), wbars[2])
    e0 = jnp.exp(wbars[0] - m)
    e1 = jnp.exp(wbars[1] - m)
    e2 = jnp.exp(wbars[2] - m)
    denom = e0 + e1 + e2
    out_ref[...] = (e0 * ys[0] + e1 * ys[1] + e2 * ys[2]) / denom + bo_ref[...]


def kernel(x, adj0, adj1, adj2, W, a, Ws, bs, q, Wo, bo):
    a3 = a.reshape(HEADS, 2 * HID, 1)
    hb, f1, f2 = pl.pallas_call(
        _prep_kernel,
        grid=(HEADS,),
        in_specs=[
            pl.BlockSpec((N, FEAT), lambda h: (0, 0)),
            pl.BlockSpec((1, FEAT, HID), lambda h: (h, 0, 0)),
            pl.BlockSpec((1, 2 * HID, 1), lambda h: (h, 0, 0)),
        ],
        out_specs=[
            pl.BlockSpec((1, N, HID + 1), lambda h: (h, 0, 0)),
            pl.BlockSpec((1, N, 1), lambda h: (h, 0, 0)),
            pl.BlockSpec((1, 1, N), lambda h: (h, 0, 0)),
        ],
        out_shape=[
            jax.ShapeDtypeStruct((HEADS, N, HID + 1), jnp.bfloat16),
            jax.ShapeDtypeStruct((HEADS, N, 1), jnp.float32),
            jax.ShapeDtypeStruct((HEADS, 1, N), jnp.float32),
        ],
    )(x, W, a3)

    adj_spec = pl.BlockSpec((BR, N), lambda i, h: (i, 0))
    z_spec = pl.BlockSpec((1, BR, HID), lambda i, h: (h, i, 0))
    z_shape = jax.ShapeDtypeStruct((HEADS, N, HID), jnp.float32)
    z0, z1, z2 = pl.pallas_call(
        _attn_kernel,
        grid=(NB, HEADS),
        in_specs=[
            adj_spec, adj_spec, adj_spec,
            pl.BlockSpec((HEADS, N, HID + 1), lambda i, h: (0, 0, 0)),
            pl.BlockSpec((HEADS, N, 1), lambda i, h: (0, 0, 0)),
            pl.BlockSpec((HEADS, 1, N), lambda i, h: (0, 0, 0)),
        ],
        out_specs=[z_spec, z_spec, z_spec],
        out_shape=[z_shape, z_shape, z_shape],
    )(adj0, adj1, adj2, hb, f1, f2)

    out = pl.pallas_call(
        _sem_kernel,
        out_shape=jax.ShapeDtypeStruct((N, CLASSES), jnp.float32),
    )(z0, z1, z2,
      Ws.reshape(HEADS, HID, QV), bs.reshape(1, QV), q.reshape(QV, 1),
      Wo.reshape(HEADS, HID, CLASSES), bo.reshape(1, CLASSES))
    return out


# speedup vs baseline: 2.8850x; 1.1436x over previous
import jax
import jax.numpy as jnp
from jax.experimental import pallas as pl

N = 2048
FEAT = 128
HID = 32
HEADS = 8
CLASSES = 16
QV = 128
ALPHA = 0.2
NEG = -9e15

BR = 512
NB = N // BR


def _prep_kernel(x_ref, W_ref, a_ref, hb_ref, f1_ref, f2_ref):
    h = jnp.dot(x_ref[...], W_ref[0], preferred_element_type=jnp.float32)
    a1 = a_ref[0, :HID, :]
    a2 = a_ref[0, HID:, :]
    hb_ref[0] = jnp.concatenate(
        [h, jnp.ones((N, 1), jnp.float32)], axis=1).astype(jnp.bfloat16)
    f1_ref[0] = jnp.dot(h, a1, preferred_element_type=jnp.float32)
    f2_ref[0] = jax.lax.dot_general(
        a2, h, (((0,), (1,)), ((), ())), preferred_element_type=jnp.float32)


def _attn_kernel(adj0_ref, adj1_ref, adj2_ref, hb_ref, f1_ref, f2_ref,
                 z0_ref, z1_ref, z2_ref):
    i = pl.program_id(0)
    hd = pl.program_id(1)
    hb = hb_ref[hd]
    f1b = f1_ref[hd, pl.ds(i * BR, BR), :]
    f2r = f2_ref[hd]
    mb = f1b + jnp.max(f2r)
    mb = jnp.where(mb >= 0, mb, ALPHA * mb)
    s = f1b + f2r
    e_raw = jnp.where(s >= 0, s, ALPHA * s)
    pshared = jnp.exp(e_raw - mb).astype(jnp.bfloat16)
    for adj_ref, z_ref in ((adj0_ref, z0_ref), (adj1_ref, z1_ref),
                           (adj2_ref, z2_ref)):
        p = pshared * adj_ref[...].astype(jnp.bfloat16)
        nd = jnp.dot(p, hb, preferred_element_type=jnp.float32)
        denom = jnp.maximum(nd[:, HID:], 1e-38)
        att = nd[:, :HID] / denom
        z_ref[0] = jnp.where(att > 0, att, jnp.exp(att) - 1.0)


def _sem_kernel(z0_ref, z1_ref, z2_ref, Ws_ref, bs_ref, q_ref, Wo_ref,
                bo_ref, out_ref):
    wbars = []
    ys = []
    for z_ref in (z0_ref, z1_ref, z2_ref):
        t = jnp.zeros((N, QV), dtype=jnp.float32)
        y = jnp.zeros((N, CLASSES), dtype=jnp.float32)
        for hd in range(HEADS):
            zh = z_ref[hd]
            t = t + jnp.dot(zh, Ws_ref[hd], preferred_element_type=jnp.float32)
            y = y + jnp.dot(zh, Wo_ref[hd], preferred_element_type=jnp.float32)
        w = jnp.dot(jnp.tanh(t + bs_ref[...]), q_ref[...],
                    preferred_element_type=jnp.float32)
        wbars.append(jnp.sum(w) / N)
        ys.append(y)
    m = jnp.maximum(jnp.maximum(wbars[0], wbars[1]), wbars[2])
    e0 = jnp.exp(wbars[0] - m)
    e1 = jnp.exp(wbars[1] - m)
    e2 = jnp.exp(wbars[2] - m)
    denom = e0 + e1 + e2
    out_ref[...] = (e0 * ys[0] + e1 * ys[1] + e2 * ys[2]) / denom + bo_ref[...]


def kernel(x, adj0, adj1, adj2, W, a, Ws, bs, q, Wo, bo):
    a3 = a.reshape(HEADS, 2 * HID, 1)
    hb, f1, f2 = pl.pallas_call(
        _prep_kernel,
        grid=(HEADS,),
        in_specs=[
            pl.BlockSpec((N, FEAT), lambda h: (0, 0)),
            pl.BlockSpec((1, FEAT, HID), lambda h: (h, 0, 0)),
            pl.BlockSpec((1, 2 * HID, 1), lambda h: (h, 0, 0)),
        ],
        out_specs=[
            pl.BlockSpec((1, N, HID + 1), lambda h: (h, 0, 0)),
            pl.BlockSpec((1, N, 1), lambda h: (h, 0, 0)),
            pl.BlockSpec((1, 1, N), lambda h: (h, 0, 0)),
        ],
        out_shape=[
            jax.ShapeDtypeStruct((HEADS, N, HID + 1), jnp.bfloat16),
            jax.ShapeDtypeStruct((HEADS, N, 1), jnp.float32),
            jax.ShapeDtypeStruct((HEADS, 1, N), jnp.float32),
        ],
    )(x, W, a3)

    adj_spec = pl.BlockSpec((BR, N), lambda i, h: (i, 0))
    z_spec = pl.BlockSpec((1, BR, HID), lambda i, h: (h, i, 0))
    z_shape = jax.ShapeDtypeStruct((HEADS, N, HID), jnp.float32)
    z0, z1, z2 = pl.pallas_call(
        _attn_kernel,
        grid=(NB, HEADS),
        in_specs=[
            adj_spec, adj_spec, adj_spec,
            pl.BlockSpec((HEADS, N, HID + 1), lambda i, h: (0, 0, 0)),
            pl.BlockSpec((HEADS, N, 1), lambda i, h: (0, 0, 0)),
            pl.BlockSpec((HEADS, 1, N), lambda i, h: (0, 0, 0)),
        ],
        out_specs=[z_spec, z_spec, z_spec],
        out_shape=[z_shape, z_shape, z_shape],
    )(adj0, adj1, adj2, hb, f1, f2)

    out = pl.pallas_call(
        _sem_kernel,
        out_shape=jax.ShapeDtypeStruct((N, CLASSES), jnp.float32),
    )(z0, z1, z2,
      Ws.reshape(HEADS, HID, QV), bs.reshape(1, QV), q.reshape(QV, 1),
      Wo.reshape(HEADS, HID, CLASSES), bo.reshape(1, CLASSES))
    return out


# speedup vs baseline: 3.0774x; 1.0667x over previous
import jax
import jax.numpy as jnp
from jax.experimental import pallas as pl
from jax.experimental.pallas import tpu as pltpu

N = 2048
FEAT = 128
HID = 32
HEADS = 8
CLASSES = 16
QV = 128
ALPHA = 0.2

BR = 512
NB = N // BR


def _han_kernel(x_ref, W_ref, a_ref, adj0_ref, adj1_ref, adj2_ref,
                Ws_ref, bs_ref, q_ref, Wo_ref, bo_ref, out_ref,
                hbt_s, f2_s, z_s):
    i = pl.program_id(0)
    hd = pl.program_id(1)

    @pl.when(i == 0)
    def _prep():
        ht = jax.lax.dot_general(
            W_ref[hd], x_ref[...], (((0,), (1,)), ((), ())),
            preferred_element_type=jnp.float32)
        a2 = a_ref[hd, HID:, :]
        f2_s[hd] = jax.lax.dot_general(
            a2, ht, (((0,), (0,)), ((), ())),
            preferred_element_type=jnp.float32)
        hbt_s[hd] = jnp.concatenate(
            [ht, jnp.ones((1, N), jnp.float32)], axis=0).astype(jnp.bfloat16)

    hbt = hbt_s[hd]
    hrows = hbt_s[hd, :HID, pl.ds(i * BR, BR)].astype(jnp.float32)
    f1b = jax.lax.dot_general(
        hrows, a_ref[hd, :HID, :], (((0,), (0,)), ((), ())),
        preferred_element_type=jnp.float32)
    f2r = f2_s[hd]
    mb = f1b + jnp.max(f2r)
    mb = jnp.where(mb >= 0, mb, ALPHA * mb)
    s = f1b + f2r
    e_raw = jnp.where(s >= 0, s, ALPHA * s)
    pshared = jnp.exp(e_raw - mb).astype(jnp.bfloat16)
    for path, adj_ref in enumerate((adj0_ref, adj1_ref, adj2_ref)):
        p = pshared * adj_ref[...].astype(jnp.bfloat16)
        nd = jax.lax.dot_general(
            p, hbt, (((1,), (1,)), ((), ())),
            preferred_element_type=jnp.float32)
        denom = jnp.maximum(nd[:, HID:], 1e-38)
        att = nd[:, :HID] / denom
        z_s[hd, pl.ds(i * BR, BR), HID * path:HID * (path + 1)] = jnp.where(
            att > 0, att, jnp.exp(att) - 1.0).astype(jnp.bfloat16)

    @pl.when(jnp.logical_and(i == NB - 1, hd == HEADS - 1))
    def _semantic():
        wbars = []
        ys = []
        for path in range(3):
            t = jnp.zeros((N, QV), dtype=jnp.float32)
            y = jnp.zeros((N, CLASSES), dtype=jnp.float32)
            for h in range(HEADS):
                zh = z_s[h, :, HID * path:HID * (path + 1)]
                t = t + jnp.dot(zh, Ws_ref[h],
                                preferred_element_type=jnp.float32)
                y = y + jnp.dot(zh, Wo_ref[h],
                                preferred_element_type=jnp.float32)
            w = jnp.dot(jnp.tanh(t + bs_ref[...]), q_ref[...],
                        preferred_element_type=jnp.float32)
            wbars.append(jnp.sum(w) / N)
            ys.append(y)
        m = jnp.maximum(jnp.maximum(wbars[0], wbars[1]), wbars[2])
        e0 = jnp.exp(wbars[0] - m)
        e1 = jnp.exp(wbars[1] - m)
        e2 = jnp.exp(wbars[2] - m)
        den = e0 + e1 + e2
        out_ref[...] = (e0 * ys[0] + e1 * ys[1] + e2 * ys[2]) / den \
            + bo_ref[...]


def kernel(x, adj0, adj1, adj2, W, a, Ws, bs, q, Wo, bo):
    adj_spec = pl.BlockSpec((BR, N), lambda i, h: (i, 0))
    res2 = lambda i, h: (0, 0)
    res3 = lambda i, h: (0, 0, 0)
    return pl.pallas_call(
        _han_kernel,
        grid=(NB, HEADS),
        in_specs=[
            pl.BlockSpec((N, FEAT), res2),
            pl.BlockSpec((HEADS, FEAT, HID), res3),
            pl.BlockSpec((HEADS, 2 * HID, 1), res3),
            adj_spec, adj_spec, adj_spec,
            pl.BlockSpec((HEADS, HID, QV), res3),
            pl.BlockSpec((1, QV), res2),
            pl.BlockSpec((QV, 1), res2),
            pl.BlockSpec((HEADS, HID, CLASSES), res3),
            pl.BlockSpec((1, CLASSES), res2),
        ],
        out_specs=pl.BlockSpec((N, CLASSES), res2),
        out_shape=jax.ShapeDtypeStruct((N, CLASSES), jnp.float32),
        scratch_shapes=[
            pltpu.VMEM((HEADS, HID + 1, N), jnp.bfloat16),
            pltpu.VMEM((HEADS, 1, N), jnp.float32),
            pltpu.VMEM((HEADS, N, 3 * HID), jnp.bfloat16),
        ],
    )(x, W, a.reshape(HEADS, 2 * HID, 1), adj0, adj1, adj2,
      Ws.astype(jnp.bfloat16).reshape(HEADS, HID, QV),
      bs.reshape(1, QV), q.reshape(QV, 1),
      Wo.astype(jnp.bfloat16).reshape(HEADS, HID, CLASSES),
      bo.reshape(1, CLASSES))
